# Initial kernel scaffold; baseline (speedup 1.0000x reference)
import jax
import jax.numpy as jnp
from jax.experimental import pallas as pl


def kernel(boxes, scores):
    raise NotImplementedError("write your pallas kernel here")


# speedup vs baseline: 23.1758x; 23.1758x over previous
import functools

import jax
import jax.numpy as jnp
from jax import lax
from jax.experimental import pallas as pl
from jax.experimental.pallas import tpu as pltpu

N = 20000
ROWS = 160
COLS = 128
NPAD = ROWS * COLS
MAX_KEEP = 300
IOU_THR = 0.7
NEG = -jnp.inf


def _nms_body(x1_ref, y1_ref, x2_ref, y2_ref, scores_ref, keep_ref, bx_ref):
    x1 = x1_ref[...]
    y1 = y1_ref[...]
    x2 = x2_ref[...]
    y2 = y2_ref[...]
    areas = (x2 - x1) * (y2 - y1)
    ridx = lax.broadcasted_iota(jnp.int32, (ROWS, COLS), 0)
    cidx = lax.broadcasted_iota(jnp.int32, (ROWS, COLS), 1)
    flat_idx = ridx * COLS + cidx

    def body(i, sc):
        m = jnp.max(sc)
        anyv = m > NEG
        cand = jnp.where(sc == m, flat_idx, jnp.int32(NPAD))
        best = jnp.min(cand)
        sel = flat_idx == best
        bx1 = jnp.sum(jnp.where(sel, x1, 0.0))
        by1 = jnp.sum(jnp.where(sel, y1, 0.0))
        bx2 = jnp.sum(jnp.where(sel, x2, 0.0))
        by2 = jnp.sum(jnp.where(sel, y2, 0.0))
        barea = (bx2 - bx1) * (by2 - by1)
        iw = jnp.minimum(bx2, x2) - jnp.maximum(bx1, x1)
        ih = jnp.minimum(by2, y2) - jnp.maximum(by1, y1)
        has = (iw > 0) & (ih > 0)
        inter = jnp.where(has, iw * ih, 0.0)
        iou = jnp.where(has, inter / (barea + areas - inter), 0.0)
        supp = (iou > IOU_THR) | sel
        sc = jnp.where(anyv & supp, NEG, sc)
        keep_ref[i] = jnp.where(anyv, best, jnp.int32(-1))
        bx_ref[i, 0] = jnp.where(anyv, bx1, 0.0)
        bx_ref[i, 1] = jnp.where(anyv, by1, 0.0)
        bx_ref[i, 2] = jnp.where(anyv, bx2, 0.0)
        bx_ref[i, 3] = jnp.where(anyv, by2, 0.0)
        return sc

    lax.fori_loop(0, MAX_KEEP, body, scores_ref[...])


@jax.jit
def kernel(boxes, scores):
    pad = NPAD - N
    x1 = jnp.pad(boxes[:, 0], (0, pad)).reshape(ROWS, COLS)
    y1 = jnp.pad(boxes[:, 1], (0, pad)).reshape(ROWS, COLS)
    x2 = jnp.pad(boxes[:, 2], (0, pad)).reshape(ROWS, COLS)
    y2 = jnp.pad(boxes[:, 3], (0, pad)).reshape(ROWS, COLS)
    sc = jnp.pad(scores, (0, pad), constant_values=NEG).reshape(ROWS, COLS)

    keep, kept_boxes = pl.pallas_call(
        _nms_body,
        in_specs=[
            pl.BlockSpec((ROWS, COLS), lambda: (0, 0)),
            pl.BlockSpec((ROWS, COLS), lambda: (0, 0)),
            pl.BlockSpec((ROWS, COLS), lambda: (0, 0)),
            pl.BlockSpec((ROWS, COLS), lambda: (0, 0)),
            pl.BlockSpec((ROWS, COLS), lambda: (0, 0)),
        ],
        out_specs=[
            pl.BlockSpec(memory_space=pltpu.SMEM),
            pl.BlockSpec(memory_space=pltpu.SMEM),
        ],
        out_shape=[
            jax.ShapeDtypeStruct((MAX_KEEP,), jnp.int32),
            jax.ShapeDtypeStruct((MAX_KEEP, 4), jnp.float32),
        ],
    )(x1, y1, x2, y2, sc)
    return kept_boxes, keep
